# baseline (speedup 1.0000x reference)
import functools

import jax
import jax.numpy as jnp
from jax import lax
from jax.experimental import pallas as pl
from jax.experimental.pallas import tpu as pltpu
from jax.experimental.pallas import tpu_sc as plsc

BATCH = 16384
HIST = 50
DIM = 32
QROW = 128
NC = 2
NS = 16
NW = NC * NS
BAGS_PER_W = BATCH // NW
CHUNK = 16
NCHUNK = BAGS_PER_W // CHUNK
ROWS = CHUNK * HIST
SCALE = 1.0 / HIST


def _emb_bag_body(idxq_hbm, off_hbm, table_hbm, out_hbm,
                  idxq_v, off_v, rows_v, out_v, sem):
    wid = lax.axis_index("s") * NC + lax.axis_index("c")
    bag_base = wid * BAGS_PER_W

    def chunk_body(c, carry):
        bag0 = bag_base + c * CHUNK
        pltpu.sync_copy(idxq_hbm.at[pl.ds(bag0, CHUNK)], idxq_v)
        pltpu.sync_copy(off_hbm.at[pl.ds(bag0, CHUNK)], off_v)
        copies = [
            pltpu.async_copy(table_hbm.at[idxq_v.at[i]], rows_v.at[pl.ds(i * HIST, HIST)], sem)
            for i in range(CHUNK)
        ]
        for cp in copies:
            cp.wait()

        def bag_body(i, carry2):
            r = i * HIST
            offs = [
                off_v[i, pl.ds(0, 16)],
                off_v[i, pl.ds(16, 16)],
                off_v[i, pl.ds(32, 16)],
                off_v[i, pl.ds(HIST - 16, 16)],
            ]
            acc0 = jnp.zeros((16,), jnp.float32)
            acc1 = jnp.zeros((16,), jnp.float32)
            for j in range(HIST):
                g, lane = (divmod(j, 16)) if j < 48 else (3, j - (HIST - 16))
                q = offs[g][lane]
                acc0 = acc0 + rows_v[r + j, pl.ds(q, 16)]
                acc1 = acc1 + rows_v[r + j, pl.ds(q + 16, 16)]
            out_v[i, pl.ds(0, 16)] = acc0 * SCALE
            out_v[i, pl.ds(16, 16)] = acc1 * SCALE
            return carry2

        lax.fori_loop(0, CHUNK, bag_body, 0)
        pltpu.sync_copy(out_v, out_hbm.at[pl.ds(bag0, CHUNK)])
        return carry

    lax.fori_loop(0, NCHUNK, chunk_body, 0)


def kernel(input, weight):
    idx = input.astype(jnp.int32)
    idxq = idx >> 2
    off = (idx & 3) << 5
    table = weight.reshape(weight.shape[0] * DIM // QROW, QROW)
    mesh = plsc.VectorSubcoreMesh(core_axis_name="c", subcore_axis_name="s")
    run = functools.partial(
        pl.kernel,
        mesh=mesh,
        compiler_params=pltpu.CompilerParams(use_tc_tiling_on_sc=True),
        out_type=jax.ShapeDtypeStruct((BATCH, DIM), jnp.float32),
        scratch_types=[
            pltpu.VMEM((CHUNK, HIST), jnp.int32),
            pltpu.VMEM((CHUNK, HIST), jnp.int32),
            pltpu.VMEM((ROWS, QROW), jnp.float32),
            pltpu.VMEM((CHUNK, DIM), jnp.float32),
            pltpu.SemaphoreType.DMA,
        ],
    )(_emb_bag_body)
    return run(idxq, off, table)
